# Initial kernel scaffold; baseline (speedup 1.0000x reference)
import jax
import jax.numpy as jnp
from jax.experimental import pallas as pl


def kernel(x, pos_table):
    raise NotImplementedError("write your pallas kernel here")


# speedup vs baseline: 1.4896x; 1.4896x over previous
import jax
import jax.numpy as jnp
from jax.experimental import pallas as pl


def _add_kernel(x_ref, p_ref, o_ref):
    o_ref[...] = x_ref[...] + p_ref[...]


def kernel(x, pos_table):
    B, L, D = x.shape
    TL = 512
    grid = (L // TL, B)
    return pl.pallas_call(
        _add_kernel,
        grid=grid,
        in_specs=[
            pl.BlockSpec((1, TL, D), lambda j, b: (b, j, 0)),
            pl.BlockSpec((TL, D), lambda j, b: (j, 0)),
        ],
        out_specs=pl.BlockSpec((1, TL, D), lambda j, b: (b, j, 0)),
        out_shape=jax.ShapeDtypeStruct((B, L, D), x.dtype),
    )(x, pos_table[:L])


# speedup vs baseline: 1.7390x; 1.1674x over previous
import jax
import jax.numpy as jnp
from jax.experimental import pallas as pl


def _add_kernel(x_ref, p_ref, o_ref):
    o_ref[...] = x_ref[...] + p_ref[...]


def kernel(x, pos_table):
    B, L, D = x.shape
    TL = 2048
    grid = (L // TL, B)
    return pl.pallas_call(
        _add_kernel,
        grid=grid,
        in_specs=[
            pl.BlockSpec((1, TL, D), lambda j, b: (b, j, 0)),
            pl.BlockSpec((TL, D), lambda j, b: (j, 0)),
        ],
        out_specs=pl.BlockSpec((1, TL, D), lambda j, b: (b, j, 0)),
        out_shape=jax.ShapeDtypeStruct((B, L, D), x.dtype),
    )(x, pos_table[:L])
